# baseline (speedup 1.0000x reference)
import jax
import jax.numpy as jnp
from jax import lax
from jax.experimental import pallas as pl
from jax.experimental.pallas import tpu as pltpu
from jax.experimental.pallas import tpu_sc as plsc

B = 4608
D = 256
V = 8192
TI = 1536
TJ = 2048
NI = B // TI
NJ = V // TJ
TJE = 1024
NE = V // TJE
NP = NJ + NE
LOSS_SCALE = 1.25 / (B * D)


def _boustro(i, jj):
    return jnp.where(i % 2 == 0, jj, NJ - 1 - jj)


def _emap(i, p):
    return _boustro(i, jnp.minimum(p, NJ - 1))


def _fused_body(x_ref, e_ref, idx_ref, loss_ref, enc_ref, et_ref,
                rmin, ridx):
    i = pl.program_id(0)
    p = pl.program_id(1)

    @pl.when(p < NJ)
    def _dist():
        j = _boustro(i, p)
        x = x_ref[...]
        e = e_ref[...]

        @pl.when(i == 0)
        def _():
            et_ref[...] = e.T

        x2 = jnp.sum(x * x, axis=1, keepdims=True)
        e2 = jnp.sum(e * e, axis=0, keepdims=True)
        mm = jnp.dot(x, e, preferred_element_type=jnp.float32)
        d = (x2 + e2) - 2.0 * mm
        m = jnp.min(d, axis=1, keepdims=True)
        cid = lax.broadcasted_iota(jnp.int32, (TI, TJ), 1) + j * TJ
        li = jnp.min(jnp.where(d == m, cid, jnp.int32(2 ** 30)), axis=1,
                     keepdims=True)

        @pl.when(p == 0)
        def _():
            rmin[...] = m
            ridx[...] = li

        @pl.when(p > 0)
        def _():
            pm, pi = rmin[...], ridx[...]
            take = (m < pm) | ((m == pm) & (li < pi))
            ridx[...] = jnp.where(take, li, pi)
            rmin[...] = jnp.where(take, m, pm)

        @pl.when(p == NJ - 1)
        def _():
            idx_ref[...] = ridx[...]
            part = jnp.sum(rmin[...], axis=(0, 1), keepdims=True)
            prev = jnp.where(i == 0, jnp.zeros_like(part), loss_ref[...])
            tot = prev + part
            loss_ref[...] = jnp.where(i == NI - 1, tot * LOSS_SCALE, tot)

    @pl.when(p >= NJ)
    def _enc():
        je = p - NJ
        cid = lax.broadcasted_iota(jnp.int32, (TI, TJE), 1) + je * TJE
        enc_ref[...] = jnp.where(cid == ridx[...], 1.0, 0.0).astype(
            jnp.float32)


_NC = 2
_NS = 16
NW = _NC * _NS
BPW = B // NW


def _gather_body(et_hbm, idx_hbm, out_hbm, idx_v, rows_v, sem):
    wid = lax.axis_index("s") * _NC + lax.axis_index("c")
    base = wid * BPW
    pltpu.sync_copy(idx_hbm.at[pl.ds(base, BPW)], idx_v)
    pltpu.async_copy(et_hbm.at[idx_v], rows_v, sem).wait()
    pltpu.sync_copy(rows_v, out_hbm.at[pl.ds(base, BPW)])


def kernel(inputs, embedding):
    flat = inputs.reshape(B, D)

    idx2d, loss, enc, et = pl.pallas_call(
        _fused_body,
        grid=(NI, NP),
        in_specs=[
            pl.BlockSpec((TI, D), lambda i, p: (i, 0)),
            pl.BlockSpec((D, TJ), lambda i, p: (0, _emap(i, p))),
        ],
        out_specs=[
            pl.BlockSpec((TI, 1), lambda i, p: (i, 0)),
            pl.BlockSpec((1, 1), lambda i, p: (0, 0)),
            pl.BlockSpec((TI, TJE),
                         lambda i, p: (i, jnp.maximum(p - NJ, 0))),
            pl.BlockSpec((TJ, D),
                         lambda i, p: (jnp.where(i == 0,
                                                 jnp.minimum(p, NJ - 1),
                                                 NJ - 1), 0)),
        ],
        out_shape=[
            jax.ShapeDtypeStruct((B, 1), jnp.int32),
            jax.ShapeDtypeStruct((1, 1), jnp.float32),
            jax.ShapeDtypeStruct((B, V), jnp.float32),
            jax.ShapeDtypeStruct((V, D), jnp.float32),
        ],
        scratch_shapes=[
            pltpu.VMEM((TI, 1), jnp.float32),
            pltpu.VMEM((TI, 1), jnp.int32),
        ],
    )(flat, embedding)

    quantized = pl.kernel(
        _gather_body,
        mesh=plsc.VectorSubcoreMesh(core_axis_name="c", subcore_axis_name="s"),
        out_type=jax.ShapeDtypeStruct((B, D), jnp.float32),
        scratch_types=[
            pltpu.VMEM((BPW,), jnp.int32),
            pltpu.VMEM((BPW, D), jnp.float32),
            pltpu.SemaphoreType.DMA,
        ],
    )(et, idx2d.reshape(B))

    return quantized.reshape(inputs.shape), loss[0, 0], enc


# speedup vs baseline: 1.0284x; 1.0284x over previous
import jax
import jax.numpy as jnp
from jax import lax
from jax.experimental import pallas as pl
from jax.experimental.pallas import tpu as pltpu
from jax.experimental.pallas import tpu_sc as plsc

B = 4608
D = 256
V = 8192
TI = 1536
TJ = 2048
NI = B // TI
NJ = V // TJ
NP = 2 * NJ
LOSS_SCALE = 1.25 / (B * D)


def _boustro(i, jj):
    return jnp.where(i % 2 == 0, jj, NJ - 1 - jj)


def _emap(i, p):
    return _boustro(i, jnp.minimum(p, NJ - 1))


def _fused_body(x_ref, e_ref, idx_ref, loss_ref, enc_ref, et_ref,
                rmin, ridx):
    i = pl.program_id(0)
    p = pl.program_id(1)

    @pl.when(p < NJ)
    def _dist():
        j = _boustro(i, p)
        x = x_ref[...]
        e = e_ref[...]

        @pl.when(i == 0)
        def _():
            et_ref[...] = e.T

        x2 = jnp.sum(x * x, axis=1, keepdims=True)
        e2 = jnp.sum(e * e, axis=0, keepdims=True)
        mm = jnp.dot(x, e, preferred_element_type=jnp.float32)
        d = (x2 + e2) - 2.0 * mm
        m = jnp.min(d, axis=1, keepdims=True)
        cid = lax.broadcasted_iota(jnp.int32, (TI, TJ), 1) + j * TJ
        li = jnp.min(jnp.where(d == m, cid, jnp.int32(2 ** 30)), axis=1,
                     keepdims=True)

        @pl.when(p == 0)
        def _():
            rmin[...] = m
            ridx[...] = li

        @pl.when(p > 0)
        def _():
            pm, pi = rmin[...], ridx[...]
            take = (m < pm) | ((m == pm) & (li < pi))
            ridx[...] = jnp.where(take, li, pi)
            rmin[...] = jnp.where(take, m, pm)

        @pl.when(p == NJ - 1)
        def _():
            idx_ref[...] = ridx[...]
            part = jnp.sum(rmin[...], axis=(0, 1), keepdims=True)
            prev = jnp.where(i == 0, jnp.zeros_like(part), loss_ref[...])
            tot = prev + part
            loss_ref[...] = jnp.where(i == NI - 1, tot * LOSS_SCALE, tot)

    @pl.when(p >= NJ)
    def _enc():
        je = p - NJ
        cid = lax.broadcasted_iota(jnp.int32, (TI, TJ), 1) + je * TJ
        enc_ref[...] = jnp.where(cid == ridx[...], 1.0, 0.0).astype(
            jnp.float32)


_NC = 2
_NS = 16
NW = _NC * _NS
BPW = B // NW


def _gather_body(et_hbm, idx_hbm, out_hbm, idx_v, rows_v, sem):
    wid = lax.axis_index("s") * _NC + lax.axis_index("c")
    base = wid * BPW
    pltpu.sync_copy(idx_hbm.at[pl.ds(base, BPW)], idx_v)
    pltpu.async_copy(et_hbm.at[idx_v], rows_v, sem).wait()
    pltpu.sync_copy(rows_v, out_hbm.at[pl.ds(base, BPW)])


def kernel(inputs, embedding):
    flat = inputs.reshape(B, D)

    idx2d, loss, enc, et = pl.pallas_call(
        _fused_body,
        grid=(NI, NP),
        in_specs=[
            pl.BlockSpec((TI, D), lambda i, p: (i, 0)),
            pl.BlockSpec((D, TJ), lambda i, p: (0, _emap(i, p))),
        ],
        out_specs=[
            pl.BlockSpec((TI, 1), lambda i, p: (i, 0)),
            pl.BlockSpec((1, 1), lambda i, p: (0, 0)),
            pl.BlockSpec((TI, TJ),
                         lambda i, p: (i, jnp.maximum(p - NJ, 0))),
            pl.BlockSpec((TJ, D),
                         lambda i, p: (jnp.where(i == 0,
                                                 jnp.minimum(p, NJ - 1),
                                                 NJ - 1), 0)),
        ],
        out_shape=[
            jax.ShapeDtypeStruct((B, 1), jnp.int32),
            jax.ShapeDtypeStruct((1, 1), jnp.float32),
            jax.ShapeDtypeStruct((B, V), jnp.float32),
            jax.ShapeDtypeStruct((V, D), jnp.float32),
        ],
        scratch_shapes=[
            pltpu.VMEM((TI, 1), jnp.float32),
            pltpu.VMEM((TI, 1), jnp.int32),
        ],
    )(flat, embedding)

    quantized = pl.kernel(
        _gather_body,
        mesh=plsc.VectorSubcoreMesh(core_axis_name="c", subcore_axis_name="s"),
        out_type=jax.ShapeDtypeStruct((B, D), jnp.float32),
        scratch_types=[
            pltpu.VMEM((BPW,), jnp.int32),
            pltpu.VMEM((BPW, D), jnp.float32),
            pltpu.SemaphoreType.DMA,
        ],
    )(et, idx2d.reshape(B))

    return quantized.reshape(inputs.shape), loss[0, 0], enc
